# baseline (speedup 1.0000x reference)
import functools

import jax
import jax.numpy as jnp
from jax import lax
from jax.experimental import pallas as pl
from jax.experimental.pallas import tpu as pltpu
from jax.experimental.pallas import tpu_sc as plsc

D = 64
B = 16384
H = 20
NC, NS, L = 2, 16, 16
NW = NC * NS
B_PER_W = B // NW
CHUNK = 32
N_CHUNK = B_PER_W // CHUNK
IDX_PER_GATHER = 128
G_PER_CHUNK = CHUNK * H // IDX_PER_GATHER
IDX_ROWS = B_PER_W * H // IDX_PER_GATHER


def _pool_body(players_hbm, table_hbm, out_hbm,
               idx_v, rows0, rows1, acc, sem0, sem1):
    wid = lax.axis_index("s") * NC + lax.axis_index("c")
    item_base = wid * B_PER_W

    pltpu.sync_copy(players_hbm.at[wid], idx_v)

    bufs = (rows0, rows1)
    sems = (sem0, sem1)

    def fire(c, buf, sem):
        descs = []
        for g in range(G_PER_CHUNK):
            descs.append(pltpu.async_copy(
                table_hbm.at[idx_v.at[c * G_PER_CHUNK + g]],
                buf.at[pl.ds(g * IDX_PER_GATHER, IDX_PER_GATHER)],
                sem))
        return descs

    def reduce_chunk(buf):
        def item_body(i, _):
            def j_body(j, accs):
                r = i * H + j
                return tuple(accs[k] + buf[r, pl.ds(k * L, L)]
                             for k in range(D // L))
            z = jnp.zeros((L,), jnp.float32)
            accs = lax.fori_loop(0, H, j_body, (z,) * (D // L))
            for k in range(D // L):
                acc[i, pl.ds(k * L, L)] = accs[k]
            return 0
        lax.fori_loop(0, CHUNK, item_body, 0)

    pending = fire(0, bufs[0], sems[0])
    for c in range(N_CHUNK):
        cur = c % 2
        if c + 1 < N_CHUNK:
            nxt_pending = fire(c + 1, bufs[1 - cur], sems[1 - cur])
        for d in pending:
            d.wait()
        reduce_chunk(bufs[cur])
        pltpu.sync_copy(acc, out_hbm.at[pl.ds(item_base + c * CHUNK, CHUNK)])
        if c + 1 < N_CHUNK:
            pending = nxt_pending


def _sc_pool(players_i32, emb_table):
    mesh = plsc.VectorSubcoreMesh(core_axis_name="c", subcore_axis_name="s")
    return pl.kernel(
        _pool_body,
        out_type=jax.ShapeDtypeStruct((B, D), jnp.float32),
        mesh=mesh,
        scratch_types=[
            pltpu.VMEM((IDX_ROWS, IDX_PER_GATHER), jnp.int32),
            pltpu.VMEM((CHUNK * H, D), jnp.float32),
            pltpu.VMEM((CHUNK * H, D), jnp.float32),
            pltpu.VMEM((CHUNK, D), jnp.float32),
            pltpu.SemaphoreType.DMA,
            pltpu.SemaphoreType.DMA,
        ],
        compiler_params=pltpu.CompilerParams(use_tc_tiling_on_sc=False),
        name="cbow_sc_pool",
    )(players_i32, emb_table)


def _head_body(pooled_ref, state_ref, stW_ref, stb_ref,
               W1_ref, b1_ref, W2_ref, b2_ref, out_ref):
    x = pooled_ref[...] * (1.0 / H)
    x += lax.dot_general(state_ref[...], stW_ref[...],
                         (((1,), (1,)), ((), ())),
                         preferred_element_type=jnp.float32)
    x += stb_ref[...]
    h = jnp.maximum(x, 0.0)
    h = lax.dot_general(h, W1_ref[...], (((1,), (1,)), ((), ())),
                        preferred_element_type=jnp.float32) + b1_ref[...]
    h = jnp.maximum(h, 0.0)
    out_ref[...] = lax.dot_general(h, W2_ref[...], (((1,), (1,)), ((), ())),
                                   preferred_element_type=jnp.float32) + b2_ref[...]


def _tc_head(pooled, state, state_W, state_b, W1, b1, W2, b2):
    blk = 2048
    grid = (B // blk,)
    full = lambda shape: pl.BlockSpec(shape, lambda i: (0,) * len(shape))
    return pl.pallas_call(
        _head_body,
        grid=grid,
        in_specs=[
            pl.BlockSpec((blk, D), lambda i: (i, 0)),
            pl.BlockSpec((blk, 3), lambda i: (i, 0)),
            full((D, 3)),
            full((1, D)),
            full((D // 2, D)),
            full((1, D // 2)),
            full((3, D // 2)),
            full((1, 3)),
        ],
        out_specs=pl.BlockSpec((blk, 3), lambda i: (i, 0)),
        out_shape=jax.ShapeDtypeStruct((B, 3), jnp.float32),
        name="cbow_tc_head",
    )(pooled, state, state_W, state_b.reshape(1, D), W1,
      b1.reshape(1, D // 2), W2, b2.reshape(1, 3))


def kernel(players, state, emb_table, state_W, state_b, W1, b1, W2, b2):
    players_i32 = players.astype(jnp.int32).reshape(NW, IDX_ROWS, IDX_PER_GATHER)
    pooled = _sc_pool(players_i32, emb_table)
    return _tc_head(pooled, state, state_W, state_b, W1, b1, W2, b2)
